# baseline (speedup 1.0000x reference)
import functools

import jax
import jax.numpy as jnp
from jax.experimental import pallas as pl
from jax.experimental.pallas import tpu as pltpu

ALPHA = 0.2
NEG = -1e30
LOG2E = 1.4426950408889634


def _proj_kernel(h_ref, w_ref, a_src_ref, a_dest_ref, wh_ref, f1_ref,
                 f2a_ref, f2b_ref, f2max_ref):
    i = pl.program_id(0)
    wh = jnp.dot(h_ref[...], w_ref[...], preferred_element_type=jnp.float32)
    wh_ref[...] = wh
    f1_ref[...] = jnp.dot(wh, a_src_ref[...], preferred_element_type=jnp.float32)
    f2 = jnp.dot(wh, a_dest_ref[...], preferred_element_type=jnp.float32)
    f2a_ref[...] = f2 * LOG2E
    f2b_ref[...] = f2 * (ALPHA * LOG2E)

    @pl.when(i == 0)
    def _init():
        f2max_ref[...] = jnp.full_like(f2max_ref, NEG)

    f2max_ref[...] = jnp.maximum(f2max_ref[...], jnp.max(f2))


def _attn_kernel(adj_ref, f1_ref, f2a_ref, f2b_ref, f2max_ref, wh_ref,
                 out_ref, a_ref, b_ref, acc_ref, *, block_n, n, f_out):
    j = pl.program_id(1)
    nj = pl.num_programs(1)

    @pl.when(j == 0)
    def _init():
        f1 = f1_ref[...]
        e = f1 + f2max_ref[0, 0]
        m = jnp.maximum(e, ALPHA * e)
        a_ref[...] = (f1 - m) * LOG2E
        b_ref[...] = (ALPHA * f1 - m) * LOG2E
        acc_ref[...] = jnp.zeros_like(acc_ref)

    def _update(sanitize):
        t = jnp.maximum(a_ref[...] + f2a_ref[...], b_ref[...] + f2b_ref[...])
        p = adj_ref[...] * jnp.exp2(t)
        if sanitize:
            col_ids = jax.lax.broadcasted_iota(
                jnp.int32, (1, block_n), 1) + j * block_n
            p = jnp.where(col_ids < n, p, 0.0)
        wh = wh_ref[pl.ds(j * block_n, block_n), :]
        acc_ref[...] += jnp.dot(p.astype(jnp.bfloat16), wh,
                                preferred_element_type=jnp.float32)

    @pl.when(j < nj - 1)
    def _body():
        _update(False)

    @pl.when(j == nj - 1)
    def _tail():
        _update(True)
        l = jnp.maximum(acc_ref[:, f_out:f_out + 1], 1e-30)
        hp = acc_ref[:, :f_out] / l
        out_ref[...] = jnp.where(hp > 0, hp, jnp.exp(hp) - 1.0)


def kernel(h, adj, W, a_src, a_dest):
    n, f_in = h.shape
    f_out = W.shape[1]

    bm1 = 1000 if n % 1000 == 0 else n
    wh, f1, f2a, f2b, f2max = pl.pallas_call(
        _proj_kernel,
        grid=(n // bm1,),
        in_specs=[
            pl.BlockSpec((bm1, f_in), lambda i: (i, 0)),
            pl.BlockSpec((f_in, f_out), lambda i: (0, 0)),
            pl.BlockSpec((f_out, 1), lambda i: (0, 0)),
            pl.BlockSpec((f_out, 1), lambda i: (0, 0)),
        ],
        out_specs=[
            pl.BlockSpec((bm1, f_out), lambda i: (i, 0)),
            pl.BlockSpec((bm1, 1), lambda i: (i, 0)),
            pl.BlockSpec((bm1, 1), lambda i: (i, 0)),
            pl.BlockSpec((bm1, 1), lambda i: (i, 0)),
            pl.BlockSpec((1, 1), lambda i: (0, 0)),
        ],
        out_shape=[
            jax.ShapeDtypeStruct((n, f_out), jnp.float32),
            jax.ShapeDtypeStruct((n, 1), jnp.float32),
            jax.ShapeDtypeStruct((n, 1), jnp.float32),
            jax.ShapeDtypeStruct((n, 1), jnp.float32),
            jax.ShapeDtypeStruct((1, 1), jnp.float32),
        ],
    )(h, W, a_src, a_dest)

    bm = 1000 if n % 1000 == 0 else n
    bn = 1024
    nj = pl.cdiv(n, bn)
    npad = nj * bn
    wh_ext = jnp.zeros((npad, 2 * f_out), jnp.bfloat16)
    wh_ext = wh_ext.at[:n, :f_out].set(wh.astype(jnp.bfloat16))
    wh_ext = wh_ext.at[:n, f_out].set(jnp.bfloat16(1.0))
    f2a_t = jnp.pad(f2a.reshape(1, n), ((0, 0), (0, npad - n)))
    f2b_t = jnp.pad(f2b.reshape(1, n), ((0, 0), (0, npad - n)))

    out = pl.pallas_call(
        functools.partial(_attn_kernel, block_n=bn, n=n, f_out=f_out),
        grid=(n // bm, nj),
        in_specs=[
            pl.BlockSpec((bm, bn), lambda i, j: (i, j)),
            pl.BlockSpec((bm, 1), lambda i, j: (i, 0)),
            pl.BlockSpec((1, bn), lambda i, j: (0, j)),
            pl.BlockSpec((1, bn), lambda i, j: (0, j)),
            pl.BlockSpec((1, 1), lambda i, j: (0, 0)),
            pl.BlockSpec((npad, 2 * f_out), lambda i, j: (0, 0)),
        ],
        out_specs=pl.BlockSpec((bm, f_out), lambda i, j: (i, 0)),
        out_shape=jax.ShapeDtypeStruct((n, f_out), jnp.float32),
        scratch_shapes=[
            pltpu.VMEM((bm, 1), jnp.float32),
            pltpu.VMEM((bm, 1), jnp.float32),
            pltpu.VMEM((bm, 2 * f_out), jnp.float32),
        ],
        compiler_params=pltpu.CompilerParams(
            dimension_semantics=("parallel", "arbitrary")),
    )(adj, f1, f2a_t, f2b_t, f2max, wh_ext)
    return out


# speedup vs baseline: 1.1885x; 1.1885x over previous
import functools

import jax
import jax.numpy as jnp
from jax.experimental import pallas as pl
from jax.experimental.pallas import tpu as pltpu

ALPHA = 0.2
NEG = -1e30
LOG2E = 1.4426950408889634


def _proj_kernel(h_ref, w_ref, a_src_ref, a_dest_t_ref, wh_ref, f1_ref,
                 f2t_ref, f2max_ref, *, block_m, n, f_out):
    i = pl.program_id(0)
    row_ids = jax.lax.broadcasted_iota(jnp.int32, (block_m, 1), 0) + i * block_m
    row_valid = row_ids < n

    wh = jnp.dot(h_ref[...], w_ref[...], preferred_element_type=jnp.float32)
    wh_ref[:, :f_out] = jnp.where(row_valid, wh, 0.0).astype(jnp.bfloat16)
    wh_ref[:, f_out:f_out + 1] = jnp.where(row_valid, 1.0, 0.0).astype(
        jnp.bfloat16)
    wh_ref[:, f_out + 1:] = jnp.zeros(
        (block_m, wh_ref.shape[1] - f_out - 1), jnp.bfloat16)

    f1_ref[...] = jnp.dot(wh, a_src_ref[...], preferred_element_type=jnp.float32)
    f2t = jax.lax.dot_general(
        a_dest_t_ref[...], wh,
        dimension_numbers=(((1,), (1,)), ((), ())),
        preferred_element_type=jnp.float32)
    f2t_ref[...] = f2t

    @pl.when(i == 0)
    def _init():
        f2max_ref[...] = jnp.full_like(f2max_ref, NEG)

    f2max_ref[...] = jnp.maximum(
        f2max_ref[...],
        jnp.max(jnp.where(row_valid.reshape(1, block_m), f2t, NEG)))


def _attn_kernel(adj_ref, f1_ref, f2t_ref, f2max_ref, wh_ref, out_ref,
                 a_ref, b_ref, acc_ref, *, block_n, n, f_out):
    j = pl.program_id(1)
    nj = pl.num_programs(1)

    @pl.when(j == 0)
    def _init():
        f1 = f1_ref[...]
        e = f1 + f2max_ref[0, 0]
        m = jnp.maximum(e, ALPHA * e)
        a_ref[...] = (f1 - m) * LOG2E
        b_ref[...] = (ALPHA * f1 - m) * LOG2E
        acc_ref[...] = jnp.zeros_like(acc_ref)

    def _update(sanitize):
        f2a = f2t_ref[...] * LOG2E
        t = jnp.maximum(a_ref[...] + f2a, b_ref[...] + ALPHA * f2a)
        p = adj_ref[...] * jnp.exp2(t)
        if sanitize:
            col_ids = jax.lax.broadcasted_iota(
                jnp.int32, (1, block_n), 1) + j * block_n
            p = jnp.where(col_ids < n, p, 0.0)
        wh = wh_ref[pl.ds(j * block_n, block_n), :]
        acc_ref[...] += jnp.dot(p.astype(jnp.bfloat16), wh,
                                preferred_element_type=jnp.float32)

    @pl.when(j < nj - 1)
    def _body():
        _update(False)

    @pl.when(j == nj - 1)
    def _tail():
        _update(True)
        l = jnp.maximum(acc_ref[:, f_out:f_out + 1], 1e-30)
        hp = acc_ref[:, :f_out] / l
        out_ref[...] = jnp.where(hp > 0, hp, jnp.exp(hp) - 1.0)


def kernel(h, adj, W, a_src, a_dest):
    n, f_in = h.shape
    f_out = W.shape[1]

    bn = 1024
    nj = pl.cdiv(n, bn)
    npad = nj * bn

    bm1 = bn
    wh_ext, f1, f2t, f2max = pl.pallas_call(
        functools.partial(_proj_kernel, block_m=bm1, n=n, f_out=f_out),
        grid=(npad // bm1,),
        in_specs=[
            pl.BlockSpec((bm1, f_in), lambda i: (i, 0)),
            pl.BlockSpec((f_in, f_out), lambda i: (0, 0)),
            pl.BlockSpec((f_out, 1), lambda i: (0, 0)),
            pl.BlockSpec((1, f_out), lambda i: (0, 0)),
        ],
        out_specs=[
            pl.BlockSpec((bm1, 2 * f_out), lambda i: (i, 0)),
            pl.BlockSpec((bm1, 1), lambda i: (i, 0)),
            pl.BlockSpec((1, bm1), lambda i: (0, i)),
            pl.BlockSpec((1, 1), lambda i: (0, 0)),
        ],
        out_shape=[
            jax.ShapeDtypeStruct((npad, 2 * f_out), jnp.bfloat16),
            jax.ShapeDtypeStruct((n, 1), jnp.float32),
            jax.ShapeDtypeStruct((1, npad), jnp.float32),
            jax.ShapeDtypeStruct((1, 1), jnp.float32),
        ],
    )(h, W, a_src, a_dest.T)

    bm = 1000 if n % 1000 == 0 else n
    out = pl.pallas_call(
        functools.partial(_attn_kernel, block_n=bn, n=n, f_out=f_out),
        grid=(n // bm, nj),
        in_specs=[
            pl.BlockSpec((bm, bn), lambda i, j: (i, j)),
            pl.BlockSpec((bm, 1), lambda i, j: (i, 0)),
            pl.BlockSpec((1, bn), lambda i, j: (0, j)),
            pl.BlockSpec((1, 1), lambda i, j: (0, 0)),
            pl.BlockSpec((npad, 2 * f_out), lambda i, j: (0, 0)),
        ],
        out_specs=pl.BlockSpec((bm, f_out), lambda i, j: (i, 0)),
        out_shape=jax.ShapeDtypeStruct((n, f_out), jnp.float32),
        scratch_shapes=[
            pltpu.VMEM((bm, 1), jnp.float32),
            pltpu.VMEM((bm, 1), jnp.float32),
            pltpu.VMEM((bm, 2 * f_out), jnp.float32),
        ],
        compiler_params=pltpu.CompilerParams(
            dimension_semantics=("parallel", "arbitrary")),
    )(adj, f1, f2t, f2max, wh_ext)
    return out
